# baseline (device time: 1539937 ns/iter reference)
import jax
import jax.numpy as jnp
from jax import lax
from jax.experimental import pallas as pl
from jax.experimental.pallas import tpu as pltpu

N_DEV = 32


def kernel(x, w_mat):
    m_full, k_per = x.shape
    _, n = w_mat.shape
    m_per = m_full // N_DEV

    def body(x_ref, w_ref, o_ref, acc_ref, send_sems, recv_sems, credit_sem):
        my = lax.axis_index("i")
        left = lax.rem(my + N_DEV - 1, N_DEV)
        right = lax.rem(my + 1, N_DEV)

        barrier_sem = pltpu.get_barrier_semaphore()
        pl.semaphore_signal(barrier_sem, inc=1, device_id=(left,),
                            device_id_type=pl.DeviceIdType.MESH)
        pl.semaphore_signal(barrier_sem, inc=1, device_id=(right,),
                            device_id_type=pl.DeviceIdType.MESH)
        pl.semaphore_wait(barrier_sem, 2)

        def partial_chunk(c):
            xs = x_ref[pl.ds(c * m_per, m_per), :]
            return jnp.dot(xs, w_ref[...], preferred_element_type=jnp.float32)

        acc_ref[0] = partial_chunk(lax.rem(my + N_DEV - 1, N_DEV))

        for s in range(N_DEV - 1):
            send_slot = s % 2
            recv_slot = (s + 1) % 2
            rdma = pltpu.make_async_remote_copy(
                src_ref=acc_ref.at[send_slot],
                dst_ref=acc_ref.at[recv_slot],
                send_sem=send_sems.at[send_slot],
                recv_sem=recv_sems.at[recv_slot],
                device_id=(right,),
                device_id_type=pl.DeviceIdType.MESH,
            )
            if s >= 1:
                pl.semaphore_wait(credit_sem, 1)
            rdma.start()
            rdma.wait_send()
            if s < N_DEV - 2:
                pl.semaphore_signal(credit_sem, inc=1, device_id=(left,),
                                    device_id_type=pl.DeviceIdType.MESH)
            rdma.wait_recv()
            c = lax.rem(my + 2 * N_DEV - 2 - s, N_DEV)
            acc_ref[recv_slot] = acc_ref[recv_slot] + partial_chunk(c)

        y = acc_ref[(N_DEV - 1) % 2]
        o_ref[...] = y * jax.nn.sigmoid(y)

    return pl.pallas_call(
        body,
        out_shape=jax.ShapeDtypeStruct((m_per, n), jnp.float32),
        in_specs=[
            pl.BlockSpec(memory_space=pltpu.VMEM),
            pl.BlockSpec(memory_space=pltpu.VMEM),
        ],
        out_specs=pl.BlockSpec(memory_space=pltpu.VMEM),
        scratch_shapes=[
            pltpu.VMEM((2, m_per, n), jnp.float32),
            pltpu.SemaphoreType.DMA((2,)),
            pltpu.SemaphoreType.DMA((2,)),
            pltpu.SemaphoreType.REGULAR,
        ],
        compiler_params=pltpu.CompilerParams(collective_id=0),
    )(x, w_mat)


# device time: 842750 ns/iter; 1.8273x vs baseline; 1.8273x over previous
import jax
import jax.numpy as jnp
from jax import lax
from jax.experimental import pallas as pl
from jax.experimental.pallas import tpu as pltpu

N_DEV = 32


def kernel(x, w_mat):
    m_full, k_per = x.shape
    _, n = w_mat.shape
    m_per = m_full // N_DEV

    def body(x_ref, w_ref, o_ref, acc_ref, send_sems, recv_sems, credit_sem):
        my = lax.axis_index("i")
        left = lax.rem(my + N_DEV - 1, N_DEV)
        right = lax.rem(my + 1, N_DEV)

        barrier_sem = pltpu.get_barrier_semaphore()
        pl.semaphore_signal(barrier_sem, inc=1, device_id=(left,),
                            device_id_type=pl.DeviceIdType.MESH)
        pl.semaphore_signal(barrier_sem, inc=1, device_id=(right,),
                            device_id_type=pl.DeviceIdType.MESH)
        pl.semaphore_wait(barrier_sem, 2)

        def partial_chunk(c):
            xs = x_ref[pl.ds(c * m_per, m_per), :]
            return jnp.dot(xs, w_ref[...], preferred_element_type=jnp.float32)

        acc_ref[0] = partial_chunk(lax.rem(my + N_DEV - 1, N_DEV)).astype(jnp.bfloat16)

        for s in range(N_DEV - 1):
            send_slot = s % 2
            recv_slot = (s + 1) % 2
            rdma = pltpu.make_async_remote_copy(
                src_ref=acc_ref.at[send_slot],
                dst_ref=acc_ref.at[recv_slot],
                send_sem=send_sems.at[send_slot],
                recv_sem=recv_sems.at[recv_slot],
                device_id=(right,),
                device_id_type=pl.DeviceIdType.MESH,
            )
            if s >= 1:
                pl.semaphore_wait(credit_sem, 1)
            rdma.start()
            rdma.wait_send()
            if s < N_DEV - 2:
                pl.semaphore_signal(credit_sem, inc=1, device_id=(left,),
                                    device_id_type=pl.DeviceIdType.MESH)
            rdma.wait_recv()
            c = lax.rem(my + 2 * N_DEV - 2 - s, N_DEV)
            acc = acc_ref[recv_slot].astype(jnp.float32) + partial_chunk(c)
            if s < N_DEV - 2:
                acc_ref[recv_slot] = acc.astype(jnp.bfloat16)
            else:
                o_ref[...] = acc * jax.nn.sigmoid(acc)

    return pl.pallas_call(
        body,
        out_shape=jax.ShapeDtypeStruct((m_per, n), jnp.float32),
        in_specs=[
            pl.BlockSpec(memory_space=pltpu.VMEM),
            pl.BlockSpec(memory_space=pltpu.VMEM),
        ],
        out_specs=pl.BlockSpec(memory_space=pltpu.VMEM),
        scratch_shapes=[
            pltpu.VMEM((2, m_per, n), jnp.bfloat16),
            pltpu.SemaphoreType.DMA((2,)),
            pltpu.SemaphoreType.DMA((2,)),
            pltpu.SemaphoreType.REGULAR,
        ],
        compiler_params=pltpu.CompilerParams(collective_id=0),
    )(x, w_mat)


# device time: 449015 ns/iter; 3.4296x vs baseline; 1.8769x over previous
import jax
import jax.numpy as jnp
from jax import lax
from jax.experimental import pallas as pl
from jax.experimental.pallas import tpu as pltpu

N_DEV = 32

RING = [0, 8, 16, 24, 25, 17, 9, 1, 2, 10, 18, 26, 29, 21, 13, 5,
        6, 14, 22, 30, 31, 23, 15, 7, 4, 12, 20, 28, 27, 19, 11, 3]
RING_INV = [0] * N_DEV
for _r, _m in enumerate(RING):
    RING_INV[_m] = _r


def kernel(x, w_mat):
    m_full, k_per = x.shape
    _, n = w_mat.shape
    m_per = m_full // N_DEV
    nh = n // 2

    m_pos = lax.axis_index("i")
    p_arr = jnp.asarray(RING, jnp.int32)
    r = jnp.asarray(RING_INV, jnp.int32)[m_pos]
    j = jnp.arange(N_DEV, dtype=jnp.int32)
    left = p_arr[jnp.mod(r - 1, N_DEV)]
    right = p_arr[jnp.mod(r + 1, N_DEV)]
    plus_chunks = p_arr[jnp.mod(r - 1 - j, N_DEV)]
    minus_chunks = p_arr[jnp.mod(r + 1 + j, N_DEV)]
    meta = jnp.concatenate([jnp.stack([left, right]), plus_chunks, minus_chunks])

    def body(meta_ref, x_ref, w_ref, o_ref,
             acc_p, acc_m, send_p, recv_p, send_m, recv_m, cred_p, cred_m):
        lft = meta_ref[0]
        rgt = meta_ref[1]

        barrier_sem = pltpu.get_barrier_semaphore()
        pl.semaphore_signal(barrier_sem, inc=1, device_id=(lft,),
                            device_id_type=pl.DeviceIdType.MESH)
        pl.semaphore_signal(barrier_sem, inc=1, device_id=(rgt,),
                            device_id_type=pl.DeviceIdType.MESH)
        pl.semaphore_wait(barrier_sem, 2)

        def partial(c, lo):
            xs = x_ref[pl.ds(c * m_per, m_per), :]
            return jnp.dot(xs, w_ref[:, lo:lo + nh],
                           preferred_element_type=jnp.float32)

        acc_p[0] = partial(meta_ref[2], 0).astype(jnp.bfloat16)
        acc_m[0] = partial(meta_ref[2 + N_DEV], nh).astype(jnp.bfloat16)

        for s in range(N_DEV - 1):
            ss = s % 2
            rs = (s + 1) % 2
            rdma_p = pltpu.make_async_remote_copy(
                src_ref=acc_p.at[ss], dst_ref=acc_p.at[rs],
                send_sem=send_p.at[ss], recv_sem=recv_p.at[rs],
                device_id=(rgt,), device_id_type=pl.DeviceIdType.MESH,
            )
            rdma_m = pltpu.make_async_remote_copy(
                src_ref=acc_m.at[ss], dst_ref=acc_m.at[rs],
                send_sem=send_m.at[ss], recv_sem=recv_m.at[rs],
                device_id=(lft,), device_id_type=pl.DeviceIdType.MESH,
            )
            if s >= 1:
                pl.semaphore_wait(cred_p, 1)
                pl.semaphore_wait(cred_m, 1)
            rdma_p.start()
            rdma_m.start()
            rdma_p.wait_send()
            rdma_m.wait_send()
            if s < N_DEV - 2:
                pl.semaphore_signal(cred_p, inc=1, device_id=(lft,),
                                    device_id_type=pl.DeviceIdType.MESH)
                pl.semaphore_signal(cred_m, inc=1, device_id=(rgt,),
                                    device_id_type=pl.DeviceIdType.MESH)
            rdma_p.wait_recv()
            rdma_m.wait_recv()
            acc = acc_p[rs].astype(jnp.float32) + partial(meta_ref[2 + s + 1], 0)
            bcc = acc_m[rs].astype(jnp.float32) + partial(
                meta_ref[2 + N_DEV + s + 1], nh)
            if s < N_DEV - 2:
                acc_p[rs] = acc.astype(jnp.bfloat16)
                acc_m[rs] = bcc.astype(jnp.bfloat16)
            else:
                o_ref[:, 0:nh] = acc * jax.nn.sigmoid(acc)
                o_ref[:, nh:n] = bcc * jax.nn.sigmoid(bcc)

    return pl.pallas_call(
        body,
        out_shape=jax.ShapeDtypeStruct((m_per, n), jnp.float32),
        in_specs=[
            pl.BlockSpec(memory_space=pltpu.SMEM),
            pl.BlockSpec(memory_space=pltpu.VMEM),
            pl.BlockSpec(memory_space=pltpu.VMEM),
        ],
        out_specs=pl.BlockSpec(memory_space=pltpu.VMEM),
        scratch_shapes=[
            pltpu.VMEM((2, m_per, nh), jnp.bfloat16),
            pltpu.VMEM((2, m_per, nh), jnp.bfloat16),
            pltpu.SemaphoreType.DMA((2,)),
            pltpu.SemaphoreType.DMA((2,)),
            pltpu.SemaphoreType.DMA((2,)),
            pltpu.SemaphoreType.DMA((2,)),
            pltpu.SemaphoreType.REGULAR,
            pltpu.SemaphoreType.REGULAR,
        ],
        compiler_params=pltpu.CompilerParams(collective_id=0),
    )(meta, x, w_mat)


# device time: 433239 ns/iter; 3.5545x vs baseline; 1.0364x over previous
import jax
import jax.numpy as jnp
from jax import lax
from jax.experimental import pallas as pl
from jax.experimental.pallas import tpu as pltpu

N_DEV = 32

RING = [0, 8, 16, 24, 25, 17, 9, 1, 2, 10, 18, 26, 29, 21, 13, 5,
        6, 14, 22, 30, 31, 23, 15, 7, 4, 12, 20, 28, 27, 19, 11, 3]
RING_INV = [0] * N_DEV
for _r, _m in enumerate(RING):
    RING_INV[_m] = _r


def kernel(x, w_mat):
    m_full, k_per = x.shape
    _, n = w_mat.shape
    m_per = m_full // N_DEV
    nh = n // 2

    m_pos = lax.axis_index("i")
    p_arr = jnp.asarray(RING, jnp.int32)
    r = jnp.asarray(RING_INV, jnp.int32)[m_pos]
    j = jnp.arange(N_DEV, dtype=jnp.int32)
    left = p_arr[jnp.mod(r - 1, N_DEV)]
    right = p_arr[jnp.mod(r + 1, N_DEV)]
    plus_chunks = p_arr[jnp.mod(r - 1 - j, N_DEV)]
    minus_chunks = p_arr[jnp.mod(r + 1 + j, N_DEV)]
    meta = jnp.concatenate([jnp.stack([left, right]), plus_chunks, minus_chunks])

    def body(meta_ref, x_ref, w_ref, o_ref,
             acc_p, acc_m, send_p, recv_p, send_m, recv_m, cred_p, cred_m):
        lft = meta_ref[0]
        rgt = meta_ref[1]

        def partial(c, lo):
            xs = x_ref[pl.ds(c * m_per, m_per), :]
            return jnp.dot(xs, w_ref[:, lo:lo + nh],
                           preferred_element_type=jnp.float32)

        acc_p[0] = partial(meta_ref[2], 0).astype(jnp.bfloat16)
        acc_m[0] = partial(meta_ref[2 + N_DEV], nh).astype(jnp.bfloat16)

        barrier_sem = pltpu.get_barrier_semaphore()
        pl.semaphore_signal(barrier_sem, inc=1, device_id=(lft,),
                            device_id_type=pl.DeviceIdType.MESH)
        pl.semaphore_signal(barrier_sem, inc=1, device_id=(rgt,),
                            device_id_type=pl.DeviceIdType.MESH)
        pl.semaphore_wait(barrier_sem, 2)

        for s in range(N_DEV - 1):
            ss = s % 2
            rs = (s + 1) % 2
            rdma_p = pltpu.make_async_remote_copy(
                src_ref=acc_p.at[ss], dst_ref=acc_p.at[rs],
                send_sem=send_p.at[ss], recv_sem=recv_p.at[rs],
                device_id=(rgt,), device_id_type=pl.DeviceIdType.MESH,
            )
            rdma_m = pltpu.make_async_remote_copy(
                src_ref=acc_m.at[ss], dst_ref=acc_m.at[rs],
                send_sem=send_m.at[ss], recv_sem=recv_m.at[rs],
                device_id=(lft,), device_id_type=pl.DeviceIdType.MESH,
            )
            if s >= 1:
                pl.semaphore_wait(cred_p, 1)
                pl.semaphore_wait(cred_m, 1)
            rdma_p.start()
            rdma_m.start()
            pp = partial(meta_ref[2 + s + 1], 0)
            pm = partial(meta_ref[2 + N_DEV + s + 1], nh)
            rdma_p.wait_send()
            rdma_m.wait_send()
            if s < N_DEV - 2:
                pl.semaphore_signal(cred_p, inc=1, device_id=(lft,),
                                    device_id_type=pl.DeviceIdType.MESH)
                pl.semaphore_signal(cred_m, inc=1, device_id=(rgt,),
                                    device_id_type=pl.DeviceIdType.MESH)
            rdma_p.wait_recv()
            rdma_m.wait_recv()
            acc = acc_p[rs].astype(jnp.float32) + pp
            bcc = acc_m[rs].astype(jnp.float32) + pm
            if s < N_DEV - 2:
                acc_p[rs] = acc.astype(jnp.bfloat16)
                acc_m[rs] = bcc.astype(jnp.bfloat16)
            else:
                o_ref[:, 0:nh] = acc * jax.nn.sigmoid(acc)
                o_ref[:, nh:n] = bcc * jax.nn.sigmoid(bcc)

    return pl.pallas_call(
        body,
        out_shape=jax.ShapeDtypeStruct((m_per, n), jnp.float32),
        in_specs=[
            pl.BlockSpec(memory_space=pltpu.SMEM),
            pl.BlockSpec(memory_space=pltpu.VMEM),
            pl.BlockSpec(memory_space=pltpu.VMEM),
        ],
        out_specs=pl.BlockSpec(memory_space=pltpu.VMEM),
        scratch_shapes=[
            pltpu.VMEM((2, m_per, nh), jnp.bfloat16),
            pltpu.VMEM((2, m_per, nh), jnp.bfloat16),
            pltpu.SemaphoreType.DMA((2,)),
            pltpu.SemaphoreType.DMA((2,)),
            pltpu.SemaphoreType.DMA((2,)),
            pltpu.SemaphoreType.DMA((2,)),
            pltpu.SemaphoreType.REGULAR,
            pltpu.SemaphoreType.REGULAR,
        ],
        compiler_params=pltpu.CompilerParams(collective_id=0),
    )(meta, x, w_mat)


# device time: 369102 ns/iter; 4.1721x vs baseline; 1.1738x over previous
import jax
import jax.numpy as jnp
from jax import lax
from jax.experimental import pallas as pl
from jax.experimental.pallas import tpu as pltpu

N_DEV = 32
N_SLOT = 4

RING = [0, 8, 16, 24, 25, 17, 9, 1, 2, 10, 18, 26, 29, 21, 13, 5,
        6, 14, 22, 30, 31, 23, 15, 7, 4, 12, 20, 28, 27, 19, 11, 3]
RING_INV = [0] * N_DEV
for _r, _m in enumerate(RING):
    RING_INV[_m] = _r


def kernel(x, w_mat):
    m_full, k_per = x.shape
    _, n = w_mat.shape
    m_per = m_full // N_DEV
    nq = n // 4

    m_pos = lax.axis_index("i")
    p_arr = jnp.asarray(RING, jnp.int32)
    r = jnp.asarray(RING_INV, jnp.int32)[m_pos]
    j = jnp.arange(N_DEV, dtype=jnp.int32)
    left = p_arr[jnp.mod(r - 1, N_DEV)]
    right = p_arr[jnp.mod(r + 1, N_DEV)]
    plus_chunks = p_arr[jnp.mod(r - 1 - j, N_DEV)]
    minus_chunks = p_arr[jnp.mod(r + 1 + j, N_DEV)]
    meta = jnp.concatenate([jnp.stack([left, right]), plus_chunks, minus_chunks])

    def body(meta_ref, x_ref, w_ref, o_ref,
             a_p0, a_p1, a_m0, a_m1,
             s_p0, r_p0, s_p1, r_p1, s_m0, r_m0, s_m1, r_m1,
             c_p0, c_p1, c_m0, c_m1):
        lft = meta_ref[0]
        rgt = meta_ref[1]

        rings = [
            (a_p0, s_p0, r_p0, c_p0, rgt, lft, 0 * nq, 2),
            (a_m0, s_m0, r_m0, c_m0, lft, rgt, 2 * nq, 2 + N_DEV),
            (a_p1, s_p1, r_p1, c_p1, rgt, lft, 1 * nq, 2),
            (a_m1, s_m1, r_m1, c_m1, lft, rgt, 3 * nq, 2 + N_DEV),
        ]

        def partial(c, lo):
            xs = x_ref[pl.ds(c * m_per, m_per), :]
            return jnp.dot(xs, w_ref[:, lo:lo + nq],
                           preferred_element_type=jnp.float32)

        def mk(ring, step):
            acc, ssem, rsem, _, tgt, _, _, _ = ring
            return pltpu.make_async_remote_copy(
                src_ref=acc.at[step % N_SLOT],
                dst_ref=acc.at[(step + 1) % N_SLOT],
                send_sem=ssem.at[step % N_SLOT],
                recv_sem=rsem.at[(step + 1) % N_SLOT],
                device_id=(tgt,), device_id_type=pl.DeviceIdType.MESH,
            )

        for acc, _, _, _, _, _, lo, cb in rings:
            acc[0] = partial(meta_ref[cb], lo).astype(jnp.bfloat16)

        barrier_sem = pltpu.get_barrier_semaphore()
        pl.semaphore_signal(barrier_sem, inc=1, device_id=(lft,),
                            device_id_type=pl.DeviceIdType.MESH)
        pl.semaphore_signal(barrier_sem, inc=1, device_id=(rgt,),
                            device_id_type=pl.DeviceIdType.MESH)
        pl.semaphore_wait(barrier_sem, 2)

        for ring in rings:
            mk(ring, 0).start()
        pp = [partial(meta_ref[cb + 1], lo)
              for _, _, _, _, _, _, lo, cb in rings]

        for s in range(N_DEV - 1):
            rslot = (s + 1) % N_SLOT
            for gi, ring in enumerate(rings):
                acc, ssem, rsem, csem, tgt, crd, lo, cb = ring
                mk(ring, s).wait_recv()
                val = acc[rslot].astype(jnp.float32) + pp[gi]
                if s == N_DEV - 2:
                    o_ref[:, lo:lo + nq] = val * jax.nn.sigmoid(val)
                else:
                    acc[rslot] = val.astype(jnp.bfloat16)
                    mk(ring, s).wait_send()
                    if s + 3 <= N_DEV - 2:
                        pl.semaphore_signal(csem, inc=1, device_id=(crd,),
                                            device_id_type=pl.DeviceIdType.MESH)
                    if s + 1 >= 3:
                        pl.semaphore_wait(csem, 1)
                    mk(ring, s + 1).start()
                    pp[gi] = partial(meta_ref[cb + s + 2], lo)
        for ring in rings:
            mk(ring, N_DEV - 2).wait_send()

    return pl.pallas_call(
        body,
        out_shape=jax.ShapeDtypeStruct((m_per, n), jnp.float32),
        in_specs=[
            pl.BlockSpec(memory_space=pltpu.SMEM),
            pl.BlockSpec(memory_space=pltpu.VMEM),
            pl.BlockSpec(memory_space=pltpu.VMEM),
        ],
        out_specs=pl.BlockSpec(memory_space=pltpu.VMEM),
        scratch_shapes=[
            pltpu.VMEM((N_SLOT, m_per, nq), jnp.bfloat16),
            pltpu.VMEM((N_SLOT, m_per, nq), jnp.bfloat16),
            pltpu.VMEM((N_SLOT, m_per, nq), jnp.bfloat16),
            pltpu.VMEM((N_SLOT, m_per, nq), jnp.bfloat16),
            pltpu.SemaphoreType.DMA((N_SLOT,)),
            pltpu.SemaphoreType.DMA((N_SLOT,)),
            pltpu.SemaphoreType.DMA((N_SLOT,)),
            pltpu.SemaphoreType.DMA((N_SLOT,)),
            pltpu.SemaphoreType.DMA((N_SLOT,)),
            pltpu.SemaphoreType.DMA((N_SLOT,)),
            pltpu.SemaphoreType.DMA((N_SLOT,)),
            pltpu.SemaphoreType.DMA((N_SLOT,)),
            pltpu.SemaphoreType.REGULAR,
            pltpu.SemaphoreType.REGULAR,
            pltpu.SemaphoreType.REGULAR,
            pltpu.SemaphoreType.REGULAR,
        ],
        compiler_params=pltpu.CompilerParams(collective_id=0),
    )(meta, x, w_mat)
